# baseline (speedup 1.0000x reference)
import functools

import jax
import jax.numpy as jnp
from jax import lax
from jax.experimental import pallas as pl
from jax.experimental.pallas import tpu as pltpu
from jax.experimental.pallas import tpu_sc as plsc

N_PLAYER = 100000
BATCH = 16384
TEAM_SIZE = 20
PITCH = 128

NC = 2
NS = 16
NW = NC * NS
B_PER_W = BATCH // NW
IDX_PER_W = B_PER_W * TEAM_SIZE
PAD_PER_W = B_PER_W * PITCH
LANES = 16
GROUPS = B_PER_W // LANES


def _sc_body(team_hbm, skill_hbm, out_hbm,
             table_sh, team_v, list_v, vals_v, out_v, sem_a, sem_b):
    sid = lax.axis_index("s")
    wid = sid * NC + lax.axis_index("c")
    cp_team = pltpu.async_copy(
        team_hbm.at[pl.ds(wid * PAD_PER_W, PAD_PER_W)], team_v, sem_b)

    @pl.when(sid == 0)
    def _stage():
        pltpu.sync_copy(skill_hbm, table_sh)

    plsc.subcore_barrier()
    cp_team.wait()

    lanes = lax.iota(jnp.int32, LANES)
    bases = []
    for j in range(5):
        p = lanes + j * LANES
        row = p // TEAM_SIZE
        bases.append(row * PITCH + (p - row * TEAM_SIZE))

    def compact(i, carry):
        off = i * (4 * PITCH)
        lst = i * (5 * LANES)
        for j in range(5):
            ids = plsc.load_gather(team_v, [bases[j] + off])
            list_v[pl.ds(lst + j * LANES, LANES)] = ids
        return carry

    lax.fori_loop(0, B_PER_W // 4, compact, 0)
    pltpu.async_copy(table_sh.at[list_v], vals_v, sem_a).wait()

    lane_off = lanes * TEAM_SIZE

    def group(g, carry):
        base = g * (LANES * TEAM_SIZE)
        acc = jnp.zeros((LANES,), jnp.float32)
        for t in range(TEAM_SIZE):
            acc = acc + plsc.load_gather(vals_v, [lane_off + (base + t)])
        out_v[pl.ds(g * LANES, LANES)] = acc
        return carry

    lax.fori_loop(0, GROUPS, group, 0)
    pltpu.sync_copy(out_v, out_hbm.at[pl.ds(wid * B_PER_W, B_PER_W)])


@functools.partial(
    pl.kernel,
    out_type=jax.ShapeDtypeStruct((BATCH,), jnp.float32),
    mesh=plsc.VectorSubcoreMesh(core_axis_name="c", subcore_axis_name="s"),
    compiler_params=pltpu.CompilerParams(needs_layout_passes=False),
    scratch_types=[
        pltpu.VMEM_SHARED((N_PLAYER,), jnp.float32),
        pltpu.VMEM((PAD_PER_W,), jnp.int32),
        pltpu.VMEM((IDX_PER_W,), jnp.int32),
        pltpu.VMEM((IDX_PER_W,), jnp.float32),
        pltpu.VMEM((B_PER_W,), jnp.float32),
        pltpu.SemaphoreType.DMA,
        pltpu.SemaphoreType.DMA,
    ],
)
def _sc_kernel(team_hbm, skill_hbm, out_hbm, *scratch):
    _sc_body(team_hbm, skill_hbm, out_hbm, *scratch)


def kernel(team, skill):
    team_pad = jnp.pad(team.astype(jnp.int32), ((0, 0), (0, PITCH - TEAM_SIZE)))
    team_flat = team_pad.reshape(-1)
    skill_flat = skill.reshape(-1)
    out = _sc_kernel(team_flat, skill_flat)
    return out.reshape(BATCH, 1, 1)


# speedup vs baseline: 1.1665x; 1.1665x over previous
import functools

import jax
import jax.numpy as jnp
from jax import lax
from jax.experimental import pallas as pl
from jax.experimental.pallas import tpu as pltpu
from jax.experimental.pallas import tpu_sc as plsc

N_PLAYER = 100000
BATCH = 16384
TEAM_SIZE = 20

NC = 2
NS = 16
NW = NC * NS
B_PER_W = BATCH // NW
IDX_PER_W = B_PER_W * TEAM_SIZE
LANES = 16
GROUPS = B_PER_W // LANES


def _sc_body(team_hbm, skill_hbm, out_hbm,
             table_sh, team_v, list_v, vals_v, out_v, sem_a, sem_b):
    sid = lax.axis_index("s")
    wid = sid * NC + lax.axis_index("c")
    cp_team = pltpu.async_copy(
        team_hbm.at[pl.ds(wid * B_PER_W, B_PER_W), :], team_v, sem_b)

    @pl.when(sid == 0)
    def _stage():
        pltpu.sync_copy(skill_hbm, table_sh)

    plsc.subcore_barrier()
    cp_team.wait()

    lanes = lax.iota(jnp.int32, LANES)
    row_bases, mem_bases = [], []
    for j in range(5):
        p = lanes + j * LANES
        row = p // TEAM_SIZE
        row_bases.append(row)
        mem_bases.append(p - row * TEAM_SIZE)

    def compact(i, carry):
        roff = i * 4
        lst = i * (5 * LANES)
        for j in range(5):
            ids = plsc.load_gather(team_v, [row_bases[j] + roff, mem_bases[j]])
            list_v[pl.ds(lst + j * LANES, LANES)] = ids
        return carry

    lax.fori_loop(0, B_PER_W // 4, compact, 0)
    pltpu.async_copy(table_sh.at[list_v], vals_v, sem_a).wait()

    lane_off = lanes * TEAM_SIZE

    def group(g, carry):
        base = g * (LANES * TEAM_SIZE)
        acc = jnp.zeros((LANES,), jnp.float32)
        for t in range(TEAM_SIZE):
            acc = acc + plsc.load_gather(vals_v, [lane_off + (base + t)])
        out_v[pl.ds(g * LANES, LANES)] = acc
        return carry

    lax.fori_loop(0, GROUPS, group, 0)
    pltpu.sync_copy(out_v, out_hbm.at[pl.ds(wid * B_PER_W, B_PER_W)])


@functools.partial(
    pl.kernel,
    out_type=jax.ShapeDtypeStruct((BATCH,), jnp.float32),
    mesh=plsc.VectorSubcoreMesh(core_axis_name="c", subcore_axis_name="s"),
    compiler_params=pltpu.CompilerParams(needs_layout_passes=False),
    scratch_types=[
        pltpu.VMEM_SHARED((N_PLAYER,), jnp.float32),
        pltpu.VMEM((B_PER_W, TEAM_SIZE), jnp.int32),
        pltpu.VMEM((IDX_PER_W,), jnp.int32),
        pltpu.VMEM((IDX_PER_W,), jnp.float32),
        pltpu.VMEM((B_PER_W,), jnp.float32),
        pltpu.SemaphoreType.DMA,
        pltpu.SemaphoreType.DMA,
    ],
)
def _sc_kernel(team_hbm, skill_hbm, out_hbm, *scratch):
    _sc_body(team_hbm, skill_hbm, out_hbm, *scratch)


def kernel(team, skill):
    out = _sc_kernel(team.astype(jnp.int32), skill.reshape(-1))
    return out.reshape(BATCH, 1, 1)


# speedup vs baseline: 1.4920x; 1.2791x over previous
import functools

import jax
import jax.numpy as jnp
from jax import lax
from jax.experimental import pallas as pl
from jax.experimental.pallas import tpu as pltpu
from jax.experimental.pallas import tpu_sc as plsc

N_PLAYER = 100000
BATCH = 16384
TEAM_SIZE = 20

NC = 2
NS = 16
NW = NC * NS
B_PER_W = BATCH // NW
IDX_PER_W = B_PER_W * TEAM_SIZE
LANES = 16
GROUPS = B_PER_W // LANES


def _sc_body(team_hbm, skill_hbm, out_hbm,
             table_sh, team_v, list_v, vals_v, out_v, sem_a, sem_b):
    sid = lax.axis_index("s")
    wid = sid * NC + lax.axis_index("c")
    cp_team = pltpu.async_copy(
        team_hbm.at[:, pl.ds(wid * B_PER_W, B_PER_W)], team_v, sem_b)

    @pl.when(sid == 0)
    def _stage():
        pltpu.sync_copy(skill_hbm, table_sh)

    plsc.subcore_barrier()
    cp_team.wait()

    def repack(i, carry):
        off = i * LANES
        for t in range(TEAM_SIZE):
            list_v[pl.ds(t * B_PER_W + off, LANES)] = team_v[t, pl.ds(off, LANES)]
        return carry

    lax.fori_loop(0, GROUPS, repack, 0)
    pltpu.async_copy(table_sh.at[list_v], vals_v, sem_a).wait()

    def group(g, carry):
        off = g * LANES
        acc = vals_v[pl.ds(off, LANES)]
        for t in range(1, TEAM_SIZE):
            acc = acc + vals_v[pl.ds(t * B_PER_W + off, LANES)]
        out_v[pl.ds(off, LANES)] = acc
        return carry

    lax.fori_loop(0, GROUPS, group, 0)
    pltpu.sync_copy(out_v, out_hbm.at[pl.ds(wid * B_PER_W, B_PER_W)])


@functools.partial(
    pl.kernel,
    out_type=jax.ShapeDtypeStruct((BATCH,), jnp.float32),
    mesh=plsc.VectorSubcoreMesh(core_axis_name="c", subcore_axis_name="s"),
    compiler_params=pltpu.CompilerParams(needs_layout_passes=False),
    scratch_types=[
        pltpu.VMEM_SHARED((N_PLAYER,), jnp.float32),
        pltpu.VMEM((TEAM_SIZE, B_PER_W), jnp.int32),
        pltpu.VMEM((IDX_PER_W,), jnp.int32),
        pltpu.VMEM((IDX_PER_W,), jnp.float32),
        pltpu.VMEM((B_PER_W,), jnp.float32),
        pltpu.SemaphoreType.DMA,
        pltpu.SemaphoreType.DMA,
    ],
)
def _sc_kernel(team_hbm, skill_hbm, out_hbm, *scratch):
    _sc_body(team_hbm, skill_hbm, out_hbm, *scratch)


def kernel(team, skill):
    out = _sc_kernel(team.astype(jnp.int32).T, skill.reshape(-1))
    return out.reshape(BATCH, 1, 1)
